# Initial kernel scaffold; baseline (speedup 1.0000x reference)
import jax
import jax.numpy as jnp
from jax.experimental import pallas as pl


def kernel(x, W1, b1, W2, b2, emb1, emb2, emb3, W3, b3, W4, b4):
    raise NotImplementedError("write your pallas kernel here")


# speedup vs baseline: 3.6677x; 3.6677x over previous
import functools

import jax
import jax.numpy as jnp
from jax.experimental import pallas as pl

_N = 4096
_IN = 768
_H = 1024
_OUT = 64
_K = 1024
_RBLK = 512
_T = 4

_HI = jax.lax.Precision.HIGHEST

_row = lambda i: (i, 0)
_rep = lambda i: (0, 0)


def _exact64(d2):
    cs = []
    for j in range(8):
        u = d2[:, 8 * j:8 * j + 4] + d2[:, 8 * j + 4:8 * j + 8]
        v = u[:, 0:2] + u[:, 2:4]
        cs.append(v[:, 0:1] + v[:, 1:2])
    acc = cs[0]
    for j in range(1, 8):
        acc = acc + cs[j]
    return acc


def _vq_select(ze, emb, esq):
    rblk = ze.shape[0]
    scores = esq - 2.0 * jax.lax.dot_general(
        ze, emb, (((1,), (1,)), ((), ())), preferred_element_type=jnp.float32)
    iota = jax.lax.broadcasted_iota(jnp.int32, (rblk, _K), 1)
    s = scores
    best_d = best_i = best_g = None
    for t in range(_T):
        it = jnp.argmin(s, axis=1).astype(jnp.int32)
        onehot = (iota == it[:, None]).astype(jnp.float32)
        g = jnp.dot(onehot, emb, precision=_HI,
                    preferred_element_type=jnp.float32)
        diff = g - ze
        dt = _exact64(diff * diff)
        it2 = it[:, None]
        if best_d is None:
            best_d, best_i, best_g = dt, it2, g
        else:
            take = (dt < best_d) | ((dt == best_d) & (it2 < best_i))
            best_d = jnp.where(take, dt, best_d)
            best_i = jnp.where(take, it2, best_i)
            best_g = jnp.where(take, g, best_g)
        if t < _T - 1:
            s = jnp.where(iota == it[:, None], jnp.inf, s)
    return best_g, best_i


def _stage_kernel(ze_ref, emb_ref, esq_ref, zq_ref, n_ref, zenext_ref):
    ze = ze_ref[...]
    zq, n = _vq_select(ze, emb_ref[...], esq_ref[...])
    zq_ref[...] = zq
    n_ref[...] = n
    zenext_ref[...] = ze - zq


def _stage3_dec_kernel(ze3_ref, emb_ref, esq_ref, ze1_ref, zq1_ref, zq2_ref,
                       W3_ref, b3_ref, W4_ref, b4_ref,
                       zq_ref, n_ref, xhat_ref):
    ze3 = ze3_ref[...]
    zq3, n3 = _vq_select(ze3, emb_ref[...], esq_ref[...])
    zq_ref[...] = zq3
    n_ref[...] = n3
    ze1 = ze1_ref[...]
    dec_in = ze1 + (((zq1_ref[...] + zq2_ref[...]) + zq3) - ze1)
    dd = jax.nn.sigmoid(
        jnp.dot(dec_in, W3_ref[...], preferred_element_type=jnp.float32)
        + b3_ref[...])
    xhat_ref[...] = (jnp.dot(dd, W4_ref[...],
                             preferred_element_type=jnp.float32)
                     + b4_ref[...])


def _mk_stage():
    return pl.pallas_call(
        _stage_kernel,
        grid=(_N // _RBLK,),
        in_specs=[
            pl.BlockSpec((_RBLK, _OUT), _row),
            pl.BlockSpec((_K, _OUT), _rep),
            pl.BlockSpec((1, _K), _rep),
        ],
        out_specs=[
            pl.BlockSpec((_RBLK, _OUT), _row),
            pl.BlockSpec((_RBLK, 1), _row),
            pl.BlockSpec((_RBLK, _OUT), _row),
        ],
        out_shape=[
            jax.ShapeDtypeStruct((_N, _OUT), jnp.float32),
            jax.ShapeDtypeStruct((_N, 1), jnp.int32),
            jax.ShapeDtypeStruct((_N, _OUT), jnp.float32),
        ],
    )


def _mk_stage3_dec():
    return pl.pallas_call(
        _stage3_dec_kernel,
        grid=(_N // _RBLK,),
        in_specs=[
            pl.BlockSpec((_RBLK, _OUT), _row),
            pl.BlockSpec((_K, _OUT), _rep),
            pl.BlockSpec((1, _K), _rep),
            pl.BlockSpec((_RBLK, _OUT), _row),
            pl.BlockSpec((_RBLK, _OUT), _row),
            pl.BlockSpec((_RBLK, _OUT), _row),
            pl.BlockSpec((_OUT, _H), _rep),
            pl.BlockSpec((1, _H), _rep),
            pl.BlockSpec((_H, _IN), _rep),
            pl.BlockSpec((1, _IN), _rep),
        ],
        out_specs=[
            pl.BlockSpec((_RBLK, _OUT), _row),
            pl.BlockSpec((_RBLK, 1), _row),
            pl.BlockSpec((_RBLK, _IN), _row),
        ],
        out_shape=[
            jax.ShapeDtypeStruct((_N, _OUT), jnp.float32),
            jax.ShapeDtypeStruct((_N, 1), jnp.int32),
            jax.ShapeDtypeStruct((_N, _IN), jnp.float32),
        ],
    )


def kernel(x, W1, b1, W2, b2, emb1, emb2, emb3, W3, b3, W4, b4):
    h = jax.nn.relu(x @ W1 + b1)
    ze_1 = h @ W2 + b2

    esq1 = jnp.sum(emb1 * emb1, axis=1).reshape(1, _K)
    esq2 = jnp.sum(emb2 * emb2, axis=1).reshape(1, _K)
    esq3 = jnp.sum(emb3 * emb3, axis=1).reshape(1, _K)

    stage = _mk_stage()
    zq_1, n1, ze_2 = stage(ze_1, emb1, esq1)
    zq_2, n2, ze_3 = stage(ze_2, emb2, esq2)
    zq_3, n3, x_hat = _mk_stage3_dec()(
        ze_3, emb3, esq3, ze_1, zq_1, zq_2,
        W3, b3.reshape(1, _H), W4, b4.reshape(1, _IN))

    return (x_hat, ze_1, ze_2, ze_3, zq_1, zq_2, zq_3,
            n1.reshape(_N), n2.reshape(_N), n3.reshape(_N))


# speedup vs baseline: 5.7108x; 1.5570x over previous
import functools

import jax
import jax.numpy as jnp
from jax.experimental import pallas as pl

_N = 4096
_IN = 768
_H = 1024
_OUT = 64
_K = 1024
_RBLK = 512
_T = 3

_HI = jax.lax.Precision.HIGHEST

_row = lambda i: (i, 0)
_rep = lambda i: (0, 0)


def _exact64(d2):
    cs = []
    for j in range(8):
        u = d2[:, 8 * j:8 * j + 4] + d2[:, 8 * j + 4:8 * j + 8]
        v = u[:, 0:2] + u[:, 2:4]
        cs.append(v[:, 0:1] + v[:, 1:2])
    acc = cs[0]
    for j in range(1, 8):
        acc = acc + cs[j]
    return acc


def _vq_select(ze, emb, esq, ea, eb, ec):
    rblk = ze.shape[0]
    scores = esq - 2.0 * jax.lax.dot_general(
        ze, emb, (((1,), (1,)), ((), ())), preferred_element_type=jnp.float32)
    iota = jax.lax.broadcasted_iota(jnp.int32, (rblk, _K), 1)
    s = scores
    best_d = best_i = best_g = None
    for t in range(_T):
        it = jnp.argmin(s, axis=1).astype(jnp.int32)
        onehot = (iota == it[:, None]).astype(jnp.bfloat16)
        ga = jnp.dot(onehot, ea, preferred_element_type=jnp.float32)
        gb = jnp.dot(onehot, eb, preferred_element_type=jnp.float32)
        gc = jnp.dot(onehot, ec, preferred_element_type=jnp.float32)
        g = (ga + gb) + gc
        diff = g - ze
        dt = _exact64(diff * diff)
        it2 = it[:, None]
        if best_d is None:
            best_d, best_i, best_g = dt, it2, g
        else:
            take = (dt < best_d) | ((dt == best_d) & (it2 < best_i))
            best_d = jnp.where(take, dt, best_d)
            best_i = jnp.where(take, it2, best_i)
            best_g = jnp.where(take, g, best_g)
        if t < _T - 1:
            s = jnp.where(iota == it[:, None], jnp.inf, s)
    return best_g, best_i


def _stage_kernel(ze_ref, emb_ref, esq_ref, ea_ref, eb_ref, ec_ref,
                  zq_ref, n_ref, zenext_ref):
    ze = ze_ref[...]
    zq, n = _vq_select(ze, emb_ref[...], esq_ref[...],
                       ea_ref[...], eb_ref[...], ec_ref[...])
    zq_ref[...] = zq
    n_ref[...] = n
    zenext_ref[...] = ze - zq


def _stage3_dec_kernel(ze3_ref, emb_ref, esq_ref, ea_ref, eb_ref, ec_ref,
                       ze1_ref, zq1_ref, zq2_ref,
                       W3_ref, b3_ref, W4_ref, b4_ref,
                       zq_ref, n_ref, xhat_ref):
    ze3 = ze3_ref[...]
    zq3, n3 = _vq_select(ze3, emb_ref[...], esq_ref[...],
                         ea_ref[...], eb_ref[...], ec_ref[...])
    zq_ref[...] = zq3
    n_ref[...] = n3
    ze1 = ze1_ref[...]
    dec_in = ze1 + (((zq1_ref[...] + zq2_ref[...]) + zq3) - ze1)
    dd = jax.nn.sigmoid(
        jnp.dot(dec_in, W3_ref[...], preferred_element_type=jnp.float32)
        + b3_ref[...])
    xhat_ref[...] = (jnp.dot(dd, W4_ref[...],
                             preferred_element_type=jnp.float32)
                     + b4_ref[...])


def _mk_stage():
    return pl.pallas_call(
        _stage_kernel,
        grid=(_N // _RBLK,),
        in_specs=[
            pl.BlockSpec((_RBLK, _OUT), _row),
            pl.BlockSpec((_K, _OUT), _rep),
            pl.BlockSpec((1, _K), _rep),
            pl.BlockSpec((_K, _OUT), _rep),
            pl.BlockSpec((_K, _OUT), _rep),
            pl.BlockSpec((_K, _OUT), _rep),
        ],
        out_specs=[
            pl.BlockSpec((_RBLK, _OUT), _row),
            pl.BlockSpec((_RBLK, 1), _row),
            pl.BlockSpec((_RBLK, _OUT), _row),
        ],
        out_shape=[
            jax.ShapeDtypeStruct((_N, _OUT), jnp.float32),
            jax.ShapeDtypeStruct((_N, 1), jnp.int32),
            jax.ShapeDtypeStruct((_N, _OUT), jnp.float32),
        ],
    )


def _mk_stage3_dec():
    return pl.pallas_call(
        _stage3_dec_kernel,
        grid=(_N // _RBLK,),
        in_specs=[
            pl.BlockSpec((_RBLK, _OUT), _row),
            pl.BlockSpec((_K, _OUT), _rep),
            pl.BlockSpec((1, _K), _rep),
            pl.BlockSpec((_K, _OUT), _rep),
            pl.BlockSpec((_K, _OUT), _rep),
            pl.BlockSpec((_K, _OUT), _rep),
            pl.BlockSpec((_RBLK, _OUT), _row),
            pl.BlockSpec((_RBLK, _OUT), _row),
            pl.BlockSpec((_RBLK, _OUT), _row),
            pl.BlockSpec((_OUT, _H), _rep),
            pl.BlockSpec((1, _H), _rep),
            pl.BlockSpec((_H, _IN), _rep),
            pl.BlockSpec((1, _IN), _rep),
        ],
        out_specs=[
            pl.BlockSpec((_RBLK, _OUT), _row),
            pl.BlockSpec((_RBLK, 1), _row),
            pl.BlockSpec((_RBLK, _IN), _row),
        ],
        out_shape=[
            jax.ShapeDtypeStruct((_N, _OUT), jnp.float32),
            jax.ShapeDtypeStruct((_N, 1), jnp.int32),
            jax.ShapeDtypeStruct((_N, _IN), jnp.float32),
        ],
    )


def kernel(x, W1, b1, W2, b2, emb1, emb2, emb3, W3, b3, W4, b4):
    h = jax.nn.relu(x @ W1 + b1)
    ze_1 = h @ W2 + b2

    esq1 = jnp.sum(emb1 * emb1, axis=1).reshape(1, _K)
    esq2 = jnp.sum(emb2 * emb2, axis=1).reshape(1, _K)
    esq3 = jnp.sum(emb3 * emb3, axis=1).reshape(1, _K)

    def split3(e):
        a = e.astype(jnp.bfloat16)
        r1 = e - a.astype(jnp.float32)
        b = r1.astype(jnp.bfloat16)
        c = (r1 - b.astype(jnp.float32)).astype(jnp.bfloat16)
        return a, b, c

    e1a, e1b, e1c = split3(emb1)
    e2a, e2b, e2c = split3(emb2)
    e3a, e3b, e3c = split3(emb3)

    stage = _mk_stage()
    zq_1, n1, ze_2 = stage(ze_1, emb1, esq1, e1a, e1b, e1c)
    zq_2, n2, ze_3 = stage(ze_2, emb2, esq2, e2a, e2b, e2c)
    zq_3, n3, x_hat = _mk_stage3_dec()(
        ze_3, emb3, esq3, e3a, e3b, e3c, ze_1, zq_1, zq_2,
        W3, b3.reshape(1, _H), W4, b4.reshape(1, _IN))

    return (x_hat, ze_1, ze_2, ze_3, zq_1, zq_2, zq_3,
            n1.reshape(_N), n2.reshape(_N), n3.reshape(_N))
